# Initial kernel scaffold; baseline (speedup 1.0000x reference)
import jax
import jax.numpy as jnp
from jax.experimental import pallas as pl


def kernel(child_feats, edge_indices, edge_type_onehot, lengths, w1, b1, w2, b2, wf, wt, wet, bed, wsec, bsec):
    raise NotImplementedError("write your pallas kernel here")


# speedup vs baseline: 1.0639x; 1.0639x over previous
import functools

import jax
import jax.numpy as jnp
from jax.experimental import pallas as pl
from jax.experimental.pallas import tpu as pltpu


def _leaky(x, slope=0.1):
    return jnp.where(x >= 0, x, slope * x)


def _gnn_kernel(NB, C, E, NI, H,
                x_ref, eidx_ref, etype_ref,
                w1_ref, b1_ref, w2_ref, b2_ref,
                wfwt_ref, wetc_ref, bed_ref,
                wsec_ref, bsec_ref, out_ref):
    f32 = jnp.float32

    x = x_ref[...]
    h = jnp.dot(x, w1_ref[...], preferred_element_type=f32) + b1_ref[...]
    h = _leaky(_leaky(h))
    h = _leaky(jnp.dot(h, w2_ref[...], preferred_element_type=f32) + b2_ref[...])

    out_acc = jnp.dot(h, wsec_ref[0], preferred_element_type=f32)

    lane = jax.lax.broadcasted_iota(jnp.int32, (2 * E, C), 1)
    ohs, scats = [], []
    for b in range(NB):
        eb = eidx_ref[b]
        gft = jnp.concatenate([eb[:, 0:1], eb[:, 1:2]], axis=0)
        oh = (lane == gft).astype(f32)
        ohs.append(oh)
        scats.append(oh[:E, :].T)

    et_all = jnp.dot(etype_ref[...], wetc_ref[...], preferred_element_type=f32)

    cur = h
    for i in range(NI):
        gs = [jnp.dot(ohs[b], cur[b * C:(b + 1) * C, :],
                      preferred_element_type=f32) for b in range(NB)]
        gf = jnp.concatenate([g[:E, :] for g in gs], axis=0)
        gt = jnp.concatenate([g[E:, :] for g in gs], axis=0)
        gcat = jnp.concatenate([gf, gt], axis=1)
        z = jnp.dot(gcat, wfwt_ref[i], preferred_element_type=f32)
        z = jnp.maximum(z + et_all[:, i * H:(i + 1) * H] + bed_ref[i], 0.0)
        cur = jnp.concatenate(
            [jnp.dot(scats[b], z[b * E:(b + 1) * E, :],
                     preferred_element_type=f32) for b in range(NB)], axis=0)
        out_acc = out_acc + jnp.dot(cur, wsec_ref[i + 1],
                                    preferred_element_type=f32)

    out_ref[...] = _leaky(out_acc + bsec_ref[...])


@jax.jit
def _forward(child_feats, edge_indices, edge_type_onehot,
             w1, b1, w2, b2, wf, wt, wet, bed, wsec, bsec):
    B, C, Fin = child_feats.shape
    E = edge_indices.shape[1]
    T = edge_type_onehot.shape[2]
    NI, _, H = wf.shape
    F_out = wsec.shape[2]
    f32 = jnp.float32

    NCORE = 2
    NB = B // NCORE

    x = child_feats.astype(f32).reshape(B * C, Fin)
    eidx = edge_indices.astype(jnp.int32)
    etype = edge_type_onehot.astype(f32).reshape(B * E, T)
    wfwt = jnp.concatenate([wf, wt], axis=1)
    wetc = jnp.concatenate(list(wet), axis=1)

    kern = functools.partial(_gnn_kernel, NB, C, E, NI, H)
    out = pl.pallas_call(
        kern,
        out_shape=jax.ShapeDtypeStruct((B * C, F_out), f32),
        grid=(NCORE,),
        in_specs=[
            pl.BlockSpec((NB * C, Fin), lambda i: (i, 0)),
            pl.BlockSpec((NB, E, 2), lambda i: (i, 0, 0)),
            pl.BlockSpec((NB * E, T), lambda i: (i, 0)),
            pl.BlockSpec((Fin, H), lambda i: (0, 0)),
            pl.BlockSpec((1, H), lambda i: (0, 0)),
            pl.BlockSpec((H, H), lambda i: (0, 0)),
            pl.BlockSpec((1, H), lambda i: (0, 0)),
            pl.BlockSpec((NI, 2 * H, H), lambda i: (0, 0, 0)),
            pl.BlockSpec((T, NI * H), lambda i: (0, 0)),
            pl.BlockSpec((NI, 1, H), lambda i: (0, 0, 0)),
            pl.BlockSpec((NI + 1, H, F_out), lambda i: (0, 0, 0)),
            pl.BlockSpec((1, F_out), lambda i: (0, 0)),
        ],
        out_specs=pl.BlockSpec((NB * C, F_out), lambda i: (i, 0)),
        compiler_params=pltpu.CompilerParams(
            dimension_semantics=("parallel",)),
    )(x, eidx, etype, w1, b1, w2, b2, wfwt, wetc, bed, wsec, bsec)
    return out.reshape(B, C, F_out)


def kernel(child_feats, edge_indices, edge_type_onehot, lengths,
           w1, b1, w2, b2, wf, wt, wet, bed, wsec, bsec):
    del lengths
    return _forward(child_feats, edge_indices, edge_type_onehot,
                    w1, b1, w2, b2, wf, wt, wet, bed, wsec, bsec)


# speedup vs baseline: 1.4919x; 1.4024x over previous
import functools

import jax
import jax.numpy as jnp
from jax.experimental import pallas as pl
from jax.experimental.pallas import tpu as pltpu


def _leaky(x, slope=0.1):
    return jnp.where(x >= 0, x, slope * x)


def _gnn_kernel(NB, C, E, NI, H,
                x_ref, eidx_ref, etype_ref,
                w1_ref, b1_ref, w2_ref, b2_ref,
                wf_ref, wt_ref, wet_ref, bed_ref,
                wsec_ref, bsec_ref, out_ref):
    f32 = jnp.float32

    x = x_ref[...]
    h = jnp.dot(x, w1_ref[...], preferred_element_type=f32) + b1_ref[...]
    h = _leaky(_leaky(h))
    h = _leaky(jnp.dot(h, w2_ref[...], preferred_element_type=f32) + b2_ref[...])

    out_acc = jnp.dot(h, wsec_ref[0], preferred_element_type=f32)

    lane = jax.lax.broadcasted_iota(jnp.int32, (2 * E, C), 1)
    ohs, scats = [], []
    for b in range(NB):
        eb = eidx_ref[b]
        gft = jnp.concatenate([eb[:, 0:1], eb[:, 1:2]], axis=0)
        oh = (lane == gft).astype(f32)
        ohs.append(oh)
        scats.append(oh[:E, :].T)

    etype = etype_ref[...]

    cur = h
    for i in range(NI):
        gs = [jnp.dot(ohs[b], cur[b * C:(b + 1) * C, :],
                      preferred_element_type=f32) for b in range(NB)]
        gf = jnp.concatenate([g[:E, :] for g in gs], axis=0)
        gt = jnp.concatenate([g[E:, :] for g in gs], axis=0)
        z = (jnp.dot(gf, wf_ref[i], preferred_element_type=f32)
             + jnp.dot(gt, wt_ref[i], preferred_element_type=f32)
             + jnp.dot(etype, wet_ref[i], preferred_element_type=f32))
        z = jnp.maximum(z + bed_ref[i], 0.0)
        cur = jnp.concatenate(
            [jnp.dot(scats[b], z[b * E:(b + 1) * E, :],
                     preferred_element_type=f32) for b in range(NB)], axis=0)
        out_acc = out_acc + jnp.dot(cur, wsec_ref[i + 1],
                                    preferred_element_type=f32)

    out_ref[...] = _leaky(out_acc + bsec_ref[...])


@jax.jit
def _forward(child_feats, edge_indices, edge_type_onehot,
             w1, b1, w2, b2, wf, wt, wet, bed, wsec, bsec):
    B, C, Fin = child_feats.shape
    E = edge_indices.shape[1]
    T = edge_type_onehot.shape[2]
    NI, _, H = wf.shape
    F_out = wsec.shape[2]
    f32 = jnp.float32

    NCORE = 2
    NB = B // NCORE

    x = child_feats.astype(f32).reshape(B * C, Fin)
    eidx = edge_indices.astype(jnp.int32)
    etype = edge_type_onehot.astype(f32).reshape(B * E, T)

    kern = functools.partial(_gnn_kernel, NB, C, E, NI, H)
    out = pl.pallas_call(
        kern,
        out_shape=jax.ShapeDtypeStruct((B * C, F_out), f32),
        grid=(NCORE,),
        in_specs=[
            pl.BlockSpec((NB * C, Fin), lambda i: (i, 0)),
            pl.BlockSpec((NB, E, 2), lambda i: (i, 0, 0)),
            pl.BlockSpec((NB * E, T), lambda i: (i, 0)),
            pl.BlockSpec((Fin, H), lambda i: (0, 0)),
            pl.BlockSpec((1, H), lambda i: (0, 0)),
            pl.BlockSpec((H, H), lambda i: (0, 0)),
            pl.BlockSpec((1, H), lambda i: (0, 0)),
            pl.BlockSpec((NI, H, H), lambda i: (0, 0, 0)),
            pl.BlockSpec((NI, H, H), lambda i: (0, 0, 0)),
            pl.BlockSpec((NI, T, H), lambda i: (0, 0, 0)),
            pl.BlockSpec((NI, 1, H), lambda i: (0, 0, 0)),
            pl.BlockSpec((NI + 1, H, F_out), lambda i: (0, 0, 0)),
            pl.BlockSpec((1, F_out), lambda i: (0, 0)),
        ],
        out_specs=pl.BlockSpec((NB * C, F_out), lambda i: (i, 0)),
        compiler_params=pltpu.CompilerParams(
            dimension_semantics=("parallel",)),
    )(x, eidx, etype, w1, b1, w2, b2, wf, wt, wet, bed, wsec, bsec)
    return out.reshape(B, C, F_out)


def kernel(child_feats, edge_indices, edge_type_onehot, lengths,
           w1, b1, w2, b2, wf, wt, wet, bed, wsec, bsec):
    del lengths
    return _forward(child_feats, edge_indices, edge_type_onehot,
                    w1, b1, w2, b2, wf, wt, wet, bed, wsec, bsec)


# speedup vs baseline: 1.5963x; 1.0699x over previous
import functools

import jax
import jax.numpy as jnp
from jax.experimental import pallas as pl
from jax.experimental.pallas import tpu as pltpu


def _leaky(x, slope=0.1):
    return jnp.where(x >= 0, x, slope * x)


def _gnn_kernel(NB, C, E, NI, H,
                x_ref, eidx_ref, etype_ref,
                w1_ref, b1_ref, w2_ref, b2_ref,
                wf_ref, wt_ref, wet_ref, bed_ref,
                wsec_ref, bsec_ref, out_ref,
                cur_ref, oh_ref, scat_ref):
    f32 = jnp.float32
    j = pl.program_id(1)

    @pl.when(j == 0)
    def _encoder_stage():
        x = x_ref[...]
        h = jnp.dot(x, w1_ref[...], preferred_element_type=f32) + b1_ref[...]
        h = _leaky(_leaky(h))
        h = _leaky(jnp.dot(h, w2_ref[...], preferred_element_type=f32)
                   + b2_ref[...])
        cur_ref[...] = h
        out_ref[...] = jnp.dot(h, wsec_ref[0], preferred_element_type=f32)

        lane = jax.lax.broadcasted_iota(jnp.int32, (2 * E, C), 1)
        for b in range(NB):
            eb = eidx_ref[b]
            gft = jnp.concatenate([eb[:, 0:1], eb[:, 1:2]], axis=0)
            oh = (lane == gft).astype(f32)
            oh_ref[b] = oh
            scat_ref[b] = oh[:E, :].T

    @pl.when(j > 0)
    def _iter_stage():
        cur = cur_ref[...]
        gs = [jnp.dot(oh_ref[b], cur[b * C:(b + 1) * C, :],
                      preferred_element_type=f32) for b in range(NB)]
        gf = jnp.concatenate([g[:E, :] for g in gs], axis=0)
        gt = jnp.concatenate([g[E:, :] for g in gs], axis=0)
        z = (jnp.dot(gf, wf_ref[0], preferred_element_type=f32)
             + jnp.dot(gt, wt_ref[0], preferred_element_type=f32)
             + jnp.dot(etype_ref[...], wet_ref[0], preferred_element_type=f32))
        z = jnp.maximum(z + bed_ref[0], 0.0)
        new_cur = jnp.concatenate(
            [jnp.dot(scat_ref[b], z[b * E:(b + 1) * E, :],
                     preferred_element_type=f32) for b in range(NB)], axis=0)
        cur_ref[...] = new_cur
        out_ref[...] = out_ref[...] + jnp.dot(
            new_cur, wsec_ref[0], preferred_element_type=f32)

    @pl.when(j == NI)
    def _finalize_stage():
        out_ref[...] = _leaky(out_ref[...] + bsec_ref[...])


@jax.jit
def _forward(child_feats, edge_indices, edge_type_onehot,
             w1, b1, w2, b2, wf, wt, wet, bed, wsec, bsec):
    B, C, Fin = child_feats.shape
    E = edge_indices.shape[1]
    T = edge_type_onehot.shape[2]
    NI, _, H = wf.shape
    F_out = wsec.shape[2]
    f32 = jnp.float32

    NCORE = 2
    NB = B // NCORE

    x = child_feats.astype(f32).reshape(B * C, Fin)
    eidx = edge_indices.astype(jnp.int32)
    etype = edge_type_onehot.astype(f32).reshape(B * E, T)

    def slab(j):
        return jnp.maximum(j - 1, 0)

    kern = functools.partial(_gnn_kernel, NB, C, E, NI, H)
    out = pl.pallas_call(
        kern,
        out_shape=jax.ShapeDtypeStruct((B * C, F_out), f32),
        grid=(NCORE, NI + 1),
        in_specs=[
            pl.BlockSpec((NB * C, Fin), lambda i, j: (i, 0)),
            pl.BlockSpec((NB, E, 2), lambda i, j: (i, 0, 0)),
            pl.BlockSpec((NB * E, T), lambda i, j: (i, 0)),
            pl.BlockSpec((Fin, H), lambda i, j: (0, 0)),
            pl.BlockSpec((1, H), lambda i, j: (0, 0)),
            pl.BlockSpec((H, H), lambda i, j: (0, 0)),
            pl.BlockSpec((1, H), lambda i, j: (0, 0)),
            pl.BlockSpec((1, H, H), lambda i, j: (slab(j), 0, 0)),
            pl.BlockSpec((1, H, H), lambda i, j: (slab(j), 0, 0)),
            pl.BlockSpec((1, T, H), lambda i, j: (slab(j), 0, 0)),
            pl.BlockSpec((1, 1, H), lambda i, j: (slab(j), 0, 0)),
            pl.BlockSpec((1, H, F_out), lambda i, j: (j, 0, 0)),
            pl.BlockSpec((1, F_out), lambda i, j: (0, 0)),
        ],
        out_specs=pl.BlockSpec((NB * C, F_out), lambda i, j: (i, 0)),
        scratch_shapes=[
            pltpu.VMEM((NB * C, H), f32),
            pltpu.VMEM((NB, 2 * E, C), f32),
            pltpu.VMEM((NB, C, E), f32),
        ],
        compiler_params=pltpu.CompilerParams(
            dimension_semantics=("parallel", "arbitrary")),
    )(x, eidx, etype, w1, b1, w2, b2, wf, wt, wet, bed, wsec, bsec)
    return out.reshape(B, C, F_out)


def kernel(child_feats, edge_indices, edge_type_onehot, lengths,
           w1, b1, w2, b2, wf, wt, wet, bed, wsec, bsec):
    del lengths
    return _forward(child_feats, edge_indices, edge_type_onehot,
                    w1, b1, w2, b2, wf, wt, wet, bed, wsec, bsec)


# speedup vs baseline: 1.7530x; 1.0982x over previous
import functools

import jax
import jax.numpy as jnp
from jax.experimental import pallas as pl
from jax.experimental.pallas import tpu as pltpu


def _leaky(x, slope=0.1):
    return jnp.where(x >= 0, x, slope * x)


def _gnn_kernel(NB, C, E, NI, H,
                x_ref, eidx_ref, etype_ref,
                w1_ref, b1_ref, w2_ref, b2_ref,
                wf_ref, wt_ref, wet_ref, bed_ref,
                wsec_ref, bsec_ref, out_ref,
                cur_ref, oh_ref, scat_ref):
    f32 = jnp.float32
    j = pl.program_id(1)

    bf16 = jnp.bfloat16

    @pl.when(j == 0)
    def _encoder_stage():
        x = x_ref[...]
        h = jnp.dot(x, w1_ref[...], preferred_element_type=f32) + b1_ref[...]
        h = _leaky(_leaky(h))
        h = _leaky(jnp.dot(h.astype(bf16), w2_ref[...].astype(bf16),
                           preferred_element_type=f32) + b2_ref[...])
        cur_ref[...] = h
        out_ref[...] = jnp.dot(h.astype(bf16), wsec_ref[0].astype(bf16),
                               preferred_element_type=f32)

        lane = jax.lax.broadcasted_iota(jnp.int32, (2 * E, C), 1)
        for b in range(NB):
            eb = eidx_ref[b]
            gft = jnp.concatenate([eb[:, 0:1], eb[:, 1:2]], axis=0)
            oh = (lane == gft).astype(f32)
            oh_ref[b] = oh
            scat_ref[b] = oh[:E, :].T

    @pl.when(j > 0)
    def _iter_stage():
        cur = cur_ref[...].astype(bf16)
        gs = [jnp.dot(oh_ref[b].astype(bf16), cur[b * C:(b + 1) * C, :],
                      preferred_element_type=f32) for b in range(NB)]
        gf = jnp.concatenate([g[:E, :] for g in gs], axis=0)
        gt = jnp.concatenate([g[E:, :] for g in gs], axis=0)
        z = (jnp.dot(gf.astype(bf16), wf_ref[0].astype(bf16),
                     preferred_element_type=f32)
             + jnp.dot(gt.astype(bf16), wt_ref[0].astype(bf16),
                       preferred_element_type=f32)
             + jnp.dot(etype_ref[...], wet_ref[0], preferred_element_type=f32))
        z = jnp.maximum(z + bed_ref[0], 0.0).astype(bf16)
        new_cur = jnp.concatenate(
            [jnp.dot(scat_ref[b].astype(bf16), z[b * E:(b + 1) * E, :],
                     preferred_element_type=f32) for b in range(NB)], axis=0)
        cur_ref[...] = new_cur
        out_ref[...] = out_ref[...] + jnp.dot(
            new_cur.astype(bf16), wsec_ref[0].astype(bf16),
            preferred_element_type=f32)

    @pl.when(j == NI)
    def _finalize_stage():
        out_ref[...] = _leaky(out_ref[...] + bsec_ref[...])


@jax.jit
def _forward(child_feats, edge_indices, edge_type_onehot,
             w1, b1, w2, b2, wf, wt, wet, bed, wsec, bsec):
    B, C, Fin = child_feats.shape
    E = edge_indices.shape[1]
    T = edge_type_onehot.shape[2]
    NI, _, H = wf.shape
    F_out = wsec.shape[2]
    f32 = jnp.float32

    NCORE = 2
    NB = B // NCORE

    x = child_feats.astype(f32).reshape(B * C, Fin)
    eidx = edge_indices.astype(jnp.int32)
    etype = edge_type_onehot.astype(f32).reshape(B * E, T)

    def slab(j):
        return jnp.maximum(j - 1, 0)

    kern = functools.partial(_gnn_kernel, NB, C, E, NI, H)
    out = pl.pallas_call(
        kern,
        out_shape=jax.ShapeDtypeStruct((B * C, F_out), f32),
        grid=(NCORE, NI + 1),
        in_specs=[
            pl.BlockSpec((NB * C, Fin), lambda i, j: (i, 0)),
            pl.BlockSpec((NB, E, 2), lambda i, j: (i, 0, 0)),
            pl.BlockSpec((NB * E, T), lambda i, j: (i, 0)),
            pl.BlockSpec((Fin, H), lambda i, j: (0, 0)),
            pl.BlockSpec((1, H), lambda i, j: (0, 0)),
            pl.BlockSpec((H, H), lambda i, j: (0, 0)),
            pl.BlockSpec((1, H), lambda i, j: (0, 0)),
            pl.BlockSpec((1, H, H), lambda i, j: (slab(j), 0, 0)),
            pl.BlockSpec((1, H, H), lambda i, j: (slab(j), 0, 0)),
            pl.BlockSpec((1, T, H), lambda i, j: (slab(j), 0, 0)),
            pl.BlockSpec((1, 1, H), lambda i, j: (slab(j), 0, 0)),
            pl.BlockSpec((1, H, F_out), lambda i, j: (j, 0, 0)),
            pl.BlockSpec((1, F_out), lambda i, j: (0, 0)),
        ],
        out_specs=pl.BlockSpec((NB * C, F_out), lambda i, j: (i, 0)),
        scratch_shapes=[
            pltpu.VMEM((NB * C, H), f32),
            pltpu.VMEM((NB, 2 * E, C), f32),
            pltpu.VMEM((NB, C, E), f32),
        ],
        compiler_params=pltpu.CompilerParams(
            dimension_semantics=("parallel", "arbitrary")),
    )(x, eidx, etype, w1, b1, w2, b2, wf, wt, wet, bed, wsec, bsec)
    return out.reshape(B, C, F_out)


def kernel(child_feats, edge_indices, edge_type_onehot, lengths,
           w1, b1, w2, b2, wf, wt, wet, bed, wsec, bsec):
    del lengths
    return _forward(child_feats, edge_indices, edge_type_onehot,
                    w1, b1, w2, b2, wf, wt, wet, bed, wsec, bsec)
